# baseline (speedup 1.0000x reference)
import functools

import jax
import jax.numpy as jnp
from jax.experimental import pallas as pl
from jax.experimental.pallas import tpu as pltpu

_B = 16384
_EMBED = 768
_HIDDEN = 512
_K = 1024
_Q = 4
_CW = 0.25
_BLK = 256

_HI = jax.lax.Precision.HIGHEST
_LO = jax.lax.Precision.DEFAULT


def _body(x_ref, ew1_ref, eb1_ref, ew2_ref, eb2_ref, cb_ref,
          dw1_ref, db1_ref, dw2_ref, db2_ref,
          recon_ref, idx_ref, rl_ref, cl_ref, csq_ref):
    i = pl.program_id(0)
    nsteps = pl.num_programs(0)

    @pl.when(i == 0)
    def _precompute():
        ones = jnp.ones((1, _HIDDEN), jnp.float32)
        for q in range(_Q):
            cb = cb_ref[q]
            csq_ref[q, 0:1, :] = jax.lax.dot_general(
                ones, cb * cb, (((1,), (1,)), ((), ())), precision=_HI)
        rl_ref[...] = jnp.zeros((1, 1), jnp.float32)
        cl_ref[...] = jnp.zeros((1, 1), jnp.float32)

    x = x_ref[...]
    h = jnp.maximum(
        jax.lax.dot(x, ew1_ref[...], precision=_LO) + eb1_ref[...], 0.0)
    z = jax.lax.dot(h, ew2_ref[...], precision=_LO) + eb2_ref[...]

    lane_q = jax.lax.broadcasted_iota(jnp.int32, (_BLK, _Q), 1)

    def _level(q, carry):
        r, qsum, commit, idx_acc = carry
        cb = cb_ref[q]
        rc = jax.lax.dot_general(
            r, cb, (((1,), (1,)), ((), ())), precision=_LO)
        rsq = jnp.sum(r * r, axis=1, keepdims=True)
        d = rsq - 2.0 * rc + csq_ref[q, 0:1, :]
        idx = jnp.argmin(d, axis=1)
        onehot = (jax.lax.broadcasted_iota(jnp.int32, (_BLK, _K), 1)
                  == idx[:, None]).astype(jnp.float32)
        quant = jax.lax.dot(onehot, cb, precision=_HI)
        diff = quant - r
        commit = commit + jnp.sum(diff * diff, keepdims=True)
        idx_acc = jnp.where(lane_q == q, idx[:, None], idx_acc)
        return r - quant, qsum + quant, commit, idx_acc

    r, qsum, commit, idx_acc = jax.lax.fori_loop(
        0, _Q,
        _level,
        (z, jnp.zeros_like(z), jnp.zeros((1, 1), jnp.float32),
         jnp.zeros((_BLK, _Q), jnp.int32)))

    idx_ref[...] = idx_acc

    a = jnp.maximum(
        jax.lax.dot(qsum, dw1_ref[...], precision=_LO) + db1_ref[...], 0.0)
    recon = jax.lax.dot(a, dw2_ref[...], precision=_LO) + db2_ref[...]
    recon_ref[...] = recon

    rdiff = recon - x
    rsum = jnp.sum(rdiff * rdiff, keepdims=True)

    rl_ref[...] += rsum
    cl_ref[...] += commit

    @pl.when(i == nsteps - 1)
    def _finalize():
        rl_ref[...] = rl_ref[...] * (1.0 / (_B * _EMBED))
        cl_ref[...] = cl_ref[...] * (_CW / (_B * _HIDDEN))


@functools.partial(jax.jit, static_argnames=())
def kernel(x, enc_w1, enc_b1, enc_w2, enc_b2, codebooks,
           dec_w1, dec_b1, dec_w2, dec_b2):
    grid = (_B // _BLK,)
    full = lambda shape: pl.BlockSpec(shape, lambda i: (0,) * len(shape))
    recon, idxs, rl, cl = pl.pallas_call(
        _body,
        grid=grid,
        in_specs=[
            pl.BlockSpec((_BLK, _EMBED), lambda i: (i, 0)),
            full((_EMBED, _HIDDEN)),
            full((1, _HIDDEN)),
            full((_HIDDEN, _HIDDEN)),
            full((1, _HIDDEN)),
            full((_Q, _K, _HIDDEN)),
            full((_HIDDEN, _HIDDEN)),
            full((1, _HIDDEN)),
            full((_HIDDEN, _EMBED)),
            full((1, _EMBED)),
        ],
        out_specs=[
            pl.BlockSpec((_BLK, _EMBED), lambda i: (i, 0)),
            pl.BlockSpec((_BLK, _Q), lambda i: (i, 0)),
            pl.BlockSpec((1, 1), lambda i: (0, 0)),
            pl.BlockSpec((1, 1), lambda i: (0, 0)),
        ],
        out_shape=[
            jax.ShapeDtypeStruct((_B, _EMBED), jnp.float32),
            jax.ShapeDtypeStruct((_B, _Q), jnp.int32),
            jax.ShapeDtypeStruct((1, 1), jnp.float32),
            jax.ShapeDtypeStruct((1, 1), jnp.float32),
        ],
        scratch_shapes=[pltpu.VMEM((_Q, 8, _K), jnp.float32)],
    )(x, enc_w1, enc_b1[None, :], enc_w2, enc_b2[None, :], codebooks,
      dec_w1, dec_b1[None, :], dec_w2, dec_b2[None, :])
    return recon, idxs, rl[0, 0], cl[0, 0]


# speedup vs baseline: 1.4144x; 1.4144x over previous
import functools

import jax
import jax.numpy as jnp
from jax.experimental import pallas as pl
from jax.experimental.pallas import tpu as pltpu

_B = 16384
_EMBED = 768
_HIDDEN = 512
_K = 1024
_Q = 4
_CW = 0.25
_BLK = 256

_HI = jax.lax.Precision.HIGHEST
_LO = jax.lax.Precision.DEFAULT


def _body(x_ref, ew1_ref, eb1_ref, ew2_ref, eb2_ref, cb_ref,
          dw1_ref, db1_ref, dw2_ref, db2_ref,
          recon_ref, idx_ref, rl_ref, cl_ref, csq_ref,
          cbh_ref, cbm_ref, cbl_ref):
    i = pl.program_id(0)
    nsteps = pl.num_programs(0)

    @pl.when(i == 0)
    def _precompute():
        ones = jnp.ones((1, _HIDDEN), jnp.float32)
        for q in range(_Q):
            cb = cb_ref[q]
            csq_ref[q, 0:1, :] = jax.lax.dot_general(
                ones, cb * cb, (((1,), (1,)), ((), ())), precision=_HI)
            hi = cb.astype(jnp.bfloat16)
            r1 = cb - hi.astype(jnp.float32)
            mid = r1.astype(jnp.bfloat16)
            lo = (r1 - mid.astype(jnp.float32)).astype(jnp.bfloat16)
            cbh_ref[q] = hi
            cbm_ref[q] = mid
            cbl_ref[q] = lo
        rl_ref[...] = jnp.zeros((1, 1), jnp.float32)
        cl_ref[...] = jnp.zeros((1, 1), jnp.float32)

    x = x_ref[...]
    h = jnp.maximum(
        jax.lax.dot(x, ew1_ref[...], precision=_LO) + eb1_ref[...], 0.0)
    z = jax.lax.dot(h, ew2_ref[...], precision=_LO) + eb2_ref[...]

    lane_q = jax.lax.broadcasted_iota(jnp.int32, (_BLK, _Q), 1)

    def _level(q, carry):
        r, qsum, commit, idx_acc = carry
        cb = cb_ref[q]
        rc = jax.lax.dot_general(
            r, cb, (((1,), (1,)), ((), ())), precision=_LO)
        rsq = jnp.sum(r * r, axis=1, keepdims=True)
        d = rsq - 2.0 * rc + csq_ref[q, 0:1, :]
        idx = jnp.argmin(d, axis=1)
        onehot = (jax.lax.broadcasted_iota(jnp.int32, (_BLK, _K), 1)
                  == idx[:, None]).astype(jnp.bfloat16)
        quant = (jax.lax.dot(onehot, cbh_ref[q],
                             preferred_element_type=jnp.float32)
                 + jax.lax.dot(onehot, cbm_ref[q],
                               preferred_element_type=jnp.float32)
                 + jax.lax.dot(onehot, cbl_ref[q],
                               preferred_element_type=jnp.float32))
        diff = quant - r
        commit = commit + jnp.sum(diff * diff, keepdims=True)
        idx_acc = jnp.where(lane_q == q, idx[:, None], idx_acc)
        return r - quant, qsum + quant, commit, idx_acc

    r, qsum, commit, idx_acc = jax.lax.fori_loop(
        0, _Q,
        _level,
        (z, jnp.zeros_like(z), jnp.zeros((1, 1), jnp.float32),
         jnp.zeros((_BLK, _Q), jnp.int32)))

    idx_ref[...] = idx_acc

    a = jnp.maximum(
        jax.lax.dot(qsum, dw1_ref[...], precision=_LO) + db1_ref[...], 0.0)
    recon = jax.lax.dot(a, dw2_ref[...], precision=_LO) + db2_ref[...]
    recon_ref[...] = recon

    rdiff = recon - x
    rsum = jnp.sum(rdiff * rdiff, keepdims=True)

    rl_ref[...] += rsum
    cl_ref[...] += commit

    @pl.when(i == nsteps - 1)
    def _finalize():
        rl_ref[...] = rl_ref[...] * (1.0 / (_B * _EMBED))
        cl_ref[...] = cl_ref[...] * (_CW / (_B * _HIDDEN))


@functools.partial(jax.jit, static_argnames=())
def kernel(x, enc_w1, enc_b1, enc_w2, enc_b2, codebooks,
           dec_w1, dec_b1, dec_w2, dec_b2):
    grid = (_B // _BLK,)
    full = lambda shape: pl.BlockSpec(shape, lambda i: (0,) * len(shape))
    recon, idxs, rl, cl = pl.pallas_call(
        _body,
        grid=grid,
        in_specs=[
            pl.BlockSpec((_BLK, _EMBED), lambda i: (i, 0)),
            full((_EMBED, _HIDDEN)),
            full((1, _HIDDEN)),
            full((_HIDDEN, _HIDDEN)),
            full((1, _HIDDEN)),
            full((_Q, _K, _HIDDEN)),
            full((_HIDDEN, _HIDDEN)),
            full((1, _HIDDEN)),
            full((_HIDDEN, _EMBED)),
            full((1, _EMBED)),
        ],
        out_specs=[
            pl.BlockSpec((_BLK, _EMBED), lambda i: (i, 0)),
            pl.BlockSpec((_BLK, _Q), lambda i: (i, 0)),
            pl.BlockSpec((1, 1), lambda i: (0, 0)),
            pl.BlockSpec((1, 1), lambda i: (0, 0)),
        ],
        out_shape=[
            jax.ShapeDtypeStruct((_B, _EMBED), jnp.float32),
            jax.ShapeDtypeStruct((_B, _Q), jnp.int32),
            jax.ShapeDtypeStruct((1, 1), jnp.float32),
            jax.ShapeDtypeStruct((1, 1), jnp.float32),
        ],
        scratch_shapes=[pltpu.VMEM((_Q, 8, _K), jnp.float32),
                        pltpu.VMEM((_Q, _K, _HIDDEN), jnp.bfloat16),
                        pltpu.VMEM((_Q, _K, _HIDDEN), jnp.bfloat16),
                        pltpu.VMEM((_Q, _K, _HIDDEN), jnp.bfloat16)],
    )(x, enc_w1, enc_b1[None, :], enc_w2, enc_b2[None, :], codebooks,
      dec_w1, dec_b1[None, :], dec_w2, dec_b2[None, :])
    return recon, idxs, rl[0, 0], cl[0, 0]


# speedup vs baseline: 1.6530x; 1.1687x over previous
import functools

import jax
import jax.numpy as jnp
from jax.experimental import pallas as pl
from jax.experimental.pallas import tpu as pltpu

_B = 16384
_EMBED = 768
_HIDDEN = 512
_K = 1024
_Q = 4
_CW = 0.25
_BLK = 512

_HI = jax.lax.Precision.HIGHEST
_LO = jax.lax.Precision.DEFAULT


def _body(x_ref, ew1_ref, eb1_ref, ew2_ref, eb2_ref, cb_ref,
          dw1_ref, db1_ref, dw2_ref, db2_ref,
          recon_ref, idx_ref, rl_ref, cl_ref, csq_ref,
          cbh_ref, cbm_ref, cbl_ref):
    i = pl.program_id(0)
    nsteps = pl.num_programs(0)

    @pl.when(i == 0)
    def _precompute():
        ones = jnp.ones((1, _HIDDEN), jnp.float32)
        for q in range(_Q):
            cb = cb_ref[q]
            csq_ref[q, 0:1, :] = jax.lax.dot_general(
                ones, cb * cb, (((1,), (1,)), ((), ())), precision=_HI)
            hi = cb.astype(jnp.bfloat16)
            r1 = cb - hi.astype(jnp.float32)
            mid = r1.astype(jnp.bfloat16)
            lo = (r1 - mid.astype(jnp.float32)).astype(jnp.bfloat16)
            cbh_ref[q] = hi
            cbm_ref[q] = mid
            cbl_ref[q] = lo
        rl_ref[...] = jnp.zeros((1, 1), jnp.float32)
        cl_ref[...] = jnp.zeros((1, 1), jnp.float32)

    x = x_ref[...]
    h = jnp.maximum(
        jax.lax.dot(x, ew1_ref[...], precision=_LO) + eb1_ref[...], 0.0)
    z = jax.lax.dot(h, ew2_ref[...], precision=_LO) + eb2_ref[...]

    lane_q = jax.lax.broadcasted_iota(jnp.int32, (_BLK, _Q), 1)

    def _level(q, carry):
        r, qsum, commit, idx_acc = carry
        cb = cb_ref[q]
        rc = jax.lax.dot_general(
            r, cb, (((1,), (1,)), ((), ())), precision=_LO)
        rsq = jnp.sum(r * r, axis=1, keepdims=True)
        d = rsq - 2.0 * rc + csq_ref[q, 0:1, :]
        idx = jnp.argmin(d, axis=1)
        onehot = (jax.lax.broadcasted_iota(jnp.int32, (_BLK, _K), 1)
                  == idx[:, None]).astype(jnp.bfloat16)
        quant = (jax.lax.dot(onehot, cbh_ref[q],
                             preferred_element_type=jnp.float32)
                 + jax.lax.dot(onehot, cbm_ref[q],
                               preferred_element_type=jnp.float32)
                 + jax.lax.dot(onehot, cbl_ref[q],
                               preferred_element_type=jnp.float32))
        diff = quant - r
        commit = commit + jnp.sum(diff * diff, keepdims=True)
        idx_acc = jnp.where(lane_q == q, idx[:, None], idx_acc)
        return r - quant, qsum + quant, commit, idx_acc

    r, qsum, commit, idx_acc = jax.lax.fori_loop(
        0, _Q,
        _level,
        (z, jnp.zeros_like(z), jnp.zeros((1, 1), jnp.float32),
         jnp.zeros((_BLK, _Q), jnp.int32)))

    idx_ref[...] = idx_acc

    a = jnp.maximum(
        jax.lax.dot(qsum, dw1_ref[...], precision=_LO) + db1_ref[...], 0.0)
    recon = jax.lax.dot(a, dw2_ref[...], precision=_LO) + db2_ref[...]
    recon_ref[...] = recon

    rdiff = recon - x
    rsum = jnp.sum(rdiff * rdiff, keepdims=True)

    rl_ref[...] += rsum
    cl_ref[...] += commit

    @pl.when(i == nsteps - 1)
    def _finalize():
        rl_ref[...] = rl_ref[...] * (1.0 / (_B * _EMBED))
        cl_ref[...] = cl_ref[...] * (_CW / (_B * _HIDDEN))


@functools.partial(jax.jit, static_argnames=())
def kernel(x, enc_w1, enc_b1, enc_w2, enc_b2, codebooks,
           dec_w1, dec_b1, dec_w2, dec_b2):
    grid = (_B // _BLK,)
    full = lambda shape: pl.BlockSpec(shape, lambda i: (0,) * len(shape))
    recon, idxs, rl, cl = pl.pallas_call(
        _body,
        grid=grid,
        in_specs=[
            pl.BlockSpec((_BLK, _EMBED), lambda i: (i, 0)),
            full((_EMBED, _HIDDEN)),
            full((1, _HIDDEN)),
            full((_HIDDEN, _HIDDEN)),
            full((1, _HIDDEN)),
            full((_Q, _K, _HIDDEN)),
            full((_HIDDEN, _HIDDEN)),
            full((1, _HIDDEN)),
            full((_HIDDEN, _EMBED)),
            full((1, _EMBED)),
        ],
        out_specs=[
            pl.BlockSpec((_BLK, _EMBED), lambda i: (i, 0)),
            pl.BlockSpec((_BLK, _Q), lambda i: (i, 0)),
            pl.BlockSpec((1, 1), lambda i: (0, 0)),
            pl.BlockSpec((1, 1), lambda i: (0, 0)),
        ],
        out_shape=[
            jax.ShapeDtypeStruct((_B, _EMBED), jnp.float32),
            jax.ShapeDtypeStruct((_B, _Q), jnp.int32),
            jax.ShapeDtypeStruct((1, 1), jnp.float32),
            jax.ShapeDtypeStruct((1, 1), jnp.float32),
        ],
        scratch_shapes=[pltpu.VMEM((_Q, 8, _K), jnp.float32),
                        pltpu.VMEM((_Q, _K, _HIDDEN), jnp.bfloat16),
                        pltpu.VMEM((_Q, _K, _HIDDEN), jnp.bfloat16),
                        pltpu.VMEM((_Q, _K, _HIDDEN), jnp.bfloat16)],
    )(x, enc_w1, enc_b1[None, :], enc_w2, enc_b2[None, :], codebooks,
      dec_w1, dec_b1[None, :], dec_w2, dec_b2[None, :])
    return recon, idxs, rl[0, 0], cl[0, 0]


# speedup vs baseline: 1.7254x; 1.0438x over previous
import functools

import jax
import jax.numpy as jnp
from jax.experimental import pallas as pl
from jax.experimental.pallas import tpu as pltpu

_B = 16384
_EMBED = 768
_HIDDEN = 512
_K = 1024
_Q = 4
_CW = 0.25
_BLK = 1024

_HI = jax.lax.Precision.HIGHEST
_LO = jax.lax.Precision.DEFAULT


def _body(x_ref, ew1_ref, eb1_ref, ew2_ref, eb2_ref, cb_ref,
          dw1_ref, db1_ref, dw2_ref, db2_ref,
          recon_ref, idx_ref, rl_ref, cl_ref, csq_ref,
          cbh_ref, cbm_ref, cbl_ref):
    i = pl.program_id(0)
    nsteps = pl.num_programs(0)

    @pl.when(i == 0)
    def _precompute():
        ones = jnp.ones((1, _HIDDEN), jnp.float32)
        for q in range(_Q):
            cb = cb_ref[q]
            csq_ref[q, 0:1, :] = jax.lax.dot_general(
                ones, cb * cb, (((1,), (1,)), ((), ())), precision=_HI)
            hi = cb.astype(jnp.bfloat16)
            r1 = cb - hi.astype(jnp.float32)
            mid = r1.astype(jnp.bfloat16)
            lo = (r1 - mid.astype(jnp.float32)).astype(jnp.bfloat16)
            cbh_ref[q] = hi
            cbm_ref[q] = mid
            cbl_ref[q] = lo
        rl_ref[...] = jnp.zeros((1, 1), jnp.float32)
        cl_ref[...] = jnp.zeros((1, 1), jnp.float32)

    x = x_ref[...]
    h = jnp.maximum(
        jax.lax.dot(x, ew1_ref[...], precision=_LO) + eb1_ref[...], 0.0)
    z = jax.lax.dot(h, ew2_ref[...], precision=_LO) + eb2_ref[...]

    lane_q = jax.lax.broadcasted_iota(jnp.int32, (_BLK, _Q), 1)

    def _level(q, carry):
        r, qsum, commit, idx_acc = carry
        cb = cb_ref[q]
        rc = jax.lax.dot_general(
            r, cb, (((1,), (1,)), ((), ())), precision=_LO)
        rsq = jnp.sum(r * r, axis=1, keepdims=True)
        d = rsq - 2.0 * rc + csq_ref[q, 0:1, :]
        idx = jnp.argmin(d, axis=1)
        onehot = (jax.lax.broadcasted_iota(jnp.int32, (_BLK, _K), 1)
                  == idx[:, None]).astype(jnp.bfloat16)
        quant = (jax.lax.dot(onehot, cbh_ref[q],
                             preferred_element_type=jnp.float32)
                 + jax.lax.dot(onehot, cbm_ref[q],
                               preferred_element_type=jnp.float32)
                 + jax.lax.dot(onehot, cbl_ref[q],
                               preferred_element_type=jnp.float32))
        diff = quant - r
        commit = commit + jnp.sum(diff * diff, keepdims=True)
        idx_acc = jnp.where(lane_q == q, idx[:, None], idx_acc)
        return r - quant, qsum + quant, commit, idx_acc

    r, qsum, commit, idx_acc = jax.lax.fori_loop(
        0, _Q,
        _level,
        (z, jnp.zeros_like(z), jnp.zeros((1, 1), jnp.float32),
         jnp.zeros((_BLK, _Q), jnp.int32)))

    idx_ref[...] = idx_acc

    a = jnp.maximum(
        jax.lax.dot(qsum, dw1_ref[...], precision=_LO) + db1_ref[...], 0.0)
    recon = jax.lax.dot(a, dw2_ref[...], precision=_LO) + db2_ref[...]
    recon_ref[...] = recon

    rdiff = recon - x
    rsum = jnp.sum(rdiff * rdiff, keepdims=True)

    rl_ref[...] += rsum
    cl_ref[...] += commit

    @pl.when(i == nsteps - 1)
    def _finalize():
        rl_ref[...] = rl_ref[...] * (1.0 / (_B * _EMBED))
        cl_ref[...] = cl_ref[...] * (_CW / (_B * _HIDDEN))


@functools.partial(jax.jit, static_argnames=())
def kernel(x, enc_w1, enc_b1, enc_w2, enc_b2, codebooks,
           dec_w1, dec_b1, dec_w2, dec_b2):
    grid = (_B // _BLK,)
    full = lambda shape: pl.BlockSpec(shape, lambda i: (0,) * len(shape))
    recon, idxs, rl, cl = pl.pallas_call(
        _body,
        grid=grid,
        in_specs=[
            pl.BlockSpec((_BLK, _EMBED), lambda i: (i, 0)),
            full((_EMBED, _HIDDEN)),
            full((1, _HIDDEN)),
            full((_HIDDEN, _HIDDEN)),
            full((1, _HIDDEN)),
            full((_Q, _K, _HIDDEN)),
            full((_HIDDEN, _HIDDEN)),
            full((1, _HIDDEN)),
            full((_HIDDEN, _EMBED)),
            full((1, _EMBED)),
        ],
        out_specs=[
            pl.BlockSpec((_BLK, _EMBED), lambda i: (i, 0)),
            pl.BlockSpec((_BLK, _Q), lambda i: (i, 0)),
            pl.BlockSpec((1, 1), lambda i: (0, 0)),
            pl.BlockSpec((1, 1), lambda i: (0, 0)),
        ],
        out_shape=[
            jax.ShapeDtypeStruct((_B, _EMBED), jnp.float32),
            jax.ShapeDtypeStruct((_B, _Q), jnp.int32),
            jax.ShapeDtypeStruct((1, 1), jnp.float32),
            jax.ShapeDtypeStruct((1, 1), jnp.float32),
        ],
        scratch_shapes=[pltpu.VMEM((_Q, 8, _K), jnp.float32),
                        pltpu.VMEM((_Q, _K, _HIDDEN), jnp.bfloat16),
                        pltpu.VMEM((_Q, _K, _HIDDEN), jnp.bfloat16),
                        pltpu.VMEM((_Q, _K, _HIDDEN), jnp.bfloat16)],
    )(x, enc_w1, enc_b1[None, :], enc_w2, enc_b2[None, :], codebooks,
      dec_w1, dec_b1[None, :], dec_w2, dec_b2[None, :])
    return recon, idxs, rl[0, 0], cl[0, 0]


# speedup vs baseline: 1.7524x; 1.0157x over previous
import functools

import jax
import jax.numpy as jnp
from jax.experimental import pallas as pl
from jax.experimental.pallas import tpu as pltpu

_B = 16384
_EMBED = 768
_HIDDEN = 512
_K = 1024
_Q = 4
_CW = 0.25
_BLK = 1024

_HI = jax.lax.Precision.HIGHEST
_LO = jax.lax.Precision.DEFAULT


def _body(x_ref, ew1_ref, eb1_ref, ew2_ref, eb2_ref, cb_ref,
          dw1_ref, db1_ref, dw2_ref, db2_ref,
          recon_ref, idx_ref, rl_ref, cl_ref, csq_ref,
          cbh_ref, cbm_ref, cbl_ref):
    i = pl.program_id(0)
    nsteps = pl.num_programs(0)

    @pl.when(i == 0)
    def _precompute():
        ones = jnp.ones((1, _HIDDEN), jnp.float32)
        for q in range(_Q):
            cb = cb_ref[q]
            csq_ref[q, 0:1, :] = jax.lax.dot_general(
                ones, cb * cb, (((1,), (1,)), ((), ())), precision=_HI)
            hi = cb.astype(jnp.bfloat16)
            r1 = cb - hi.astype(jnp.float32)
            mid = r1.astype(jnp.bfloat16)
            lo = (r1 - mid.astype(jnp.float32)).astype(jnp.bfloat16)
            cbh_ref[q] = hi
            cbm_ref[q] = mid
            cbl_ref[q] = lo
        rl_ref[...] = jnp.zeros((1, 1), jnp.float32)
        cl_ref[...] = jnp.zeros((1, 1), jnp.float32)

    x = x_ref[...]
    h = jnp.maximum(
        jax.lax.dot(x, ew1_ref[...], precision=_LO) + eb1_ref[...], 0.0)
    z = jax.lax.dot(h, ew2_ref[...], precision=_LO) + eb2_ref[...]

    _HB = _BLK // 2
    lane_q = jax.lax.broadcasted_iota(jnp.int32, (_HB, _Q), 1)

    def _half(q, r, qsum, commit, idx_acc):
        cb = cb_ref[q]
        rc = jax.lax.dot_general(
            r, cb, (((1,), (1,)), ((), ())), precision=_LO)
        rsq = jnp.sum(r * r, axis=1, keepdims=True)
        d = rsq - 2.0 * rc + csq_ref[q, 0:1, :]
        idx = jnp.argmin(d, axis=1)
        onehot = (jax.lax.broadcasted_iota(jnp.int32, (_HB, _K), 1)
                  == idx[:, None]).astype(jnp.bfloat16)
        quant = (jax.lax.dot(onehot, cbh_ref[q],
                             preferred_element_type=jnp.float32)
                 + jax.lax.dot(onehot, cbm_ref[q],
                               preferred_element_type=jnp.float32)
                 + jax.lax.dot(onehot, cbl_ref[q],
                               preferred_element_type=jnp.float32))
        diff = quant - r
        commit = commit + jnp.sum(diff * diff, keepdims=True)
        idx_acc = jnp.where(lane_q == q, idx[:, None], idx_acc)
        return r - quant, qsum + quant, commit, idx_acc

    def _level(q, carry):
        ra, rb, qsa, qsb, commit, ia, ib = carry
        ra, qsa, commit, ia = _half(q, ra, qsa, commit, ia)
        rb, qsb, commit, ib = _half(q, rb, qsb, commit, ib)
        return ra, rb, qsa, qsb, commit, ia, ib

    za, zb = z[:_HB, :], z[_HB:, :]
    zero_h = jnp.zeros((_HB, _HIDDEN), jnp.float32)
    zero_i = jnp.zeros((_HB, _Q), jnp.int32)
    ra, rb, qsa, qsb, commit, ia, ib = jax.lax.fori_loop(
        0, _Q,
        _level,
        (za, zb, zero_h, zero_h, jnp.zeros((1, 1), jnp.float32),
         zero_i, zero_i))
    qsum = jnp.concatenate([qsa, qsb], axis=0)

    idx_ref[...] = jnp.concatenate([ia, ib], axis=0)

    a = jnp.maximum(
        jax.lax.dot(qsum, dw1_ref[...], precision=_LO) + db1_ref[...], 0.0)
    recon = jax.lax.dot(a, dw2_ref[...], precision=_LO) + db2_ref[...]
    recon_ref[...] = recon

    rdiff = recon - x
    rsum = jnp.sum(rdiff * rdiff, keepdims=True)

    rl_ref[...] += rsum
    cl_ref[...] += commit

    @pl.when(i == nsteps - 1)
    def _finalize():
        rl_ref[...] = rl_ref[...] * (1.0 / (_B * _EMBED))
        cl_ref[...] = cl_ref[...] * (_CW / (_B * _HIDDEN))


@functools.partial(jax.jit, static_argnames=())
def kernel(x, enc_w1, enc_b1, enc_w2, enc_b2, codebooks,
           dec_w1, dec_b1, dec_w2, dec_b2):
    grid = (_B // _BLK,)
    full = lambda shape: pl.BlockSpec(shape, lambda i: (0,) * len(shape))
    recon, idxs, rl, cl = pl.pallas_call(
        _body,
        grid=grid,
        in_specs=[
            pl.BlockSpec((_BLK, _EMBED), lambda i: (i, 0)),
            full((_EMBED, _HIDDEN)),
            full((1, _HIDDEN)),
            full((_HIDDEN, _HIDDEN)),
            full((1, _HIDDEN)),
            full((_Q, _K, _HIDDEN)),
            full((_HIDDEN, _HIDDEN)),
            full((1, _HIDDEN)),
            full((_HIDDEN, _EMBED)),
            full((1, _EMBED)),
        ],
        out_specs=[
            pl.BlockSpec((_BLK, _EMBED), lambda i: (i, 0)),
            pl.BlockSpec((_BLK, _Q), lambda i: (i, 0)),
            pl.BlockSpec((1, 1), lambda i: (0, 0)),
            pl.BlockSpec((1, 1), lambda i: (0, 0)),
        ],
        out_shape=[
            jax.ShapeDtypeStruct((_B, _EMBED), jnp.float32),
            jax.ShapeDtypeStruct((_B, _Q), jnp.int32),
            jax.ShapeDtypeStruct((1, 1), jnp.float32),
            jax.ShapeDtypeStruct((1, 1), jnp.float32),
        ],
        scratch_shapes=[pltpu.VMEM((_Q, 8, _K), jnp.float32),
                        pltpu.VMEM((_Q, _K, _HIDDEN), jnp.bfloat16),
                        pltpu.VMEM((_Q, _K, _HIDDEN), jnp.bfloat16),
                        pltpu.VMEM((_Q, _K, _HIDDEN), jnp.bfloat16)],
    )(x, enc_w1, enc_b1[None, :], enc_w2, enc_b2[None, :], codebooks,
      dec_w1, dec_b1[None, :], dec_w2, dec_b2[None, :])
    return recon, idxs, rl[0, 0], cl[0, 0]


# speedup vs baseline: 2.0011x; 1.1419x over previous
import functools

import jax
import jax.numpy as jnp
from jax.experimental import pallas as pl
from jax.experimental.pallas import tpu as pltpu

_B = 16384
_EMBED = 768
_HIDDEN = 512
_K = 1024
_Q = 4
_CW = 0.25
_BLK = 1024

_HI = jax.lax.Precision.HIGHEST
_LO = jax.lax.Precision.DEFAULT


def _body(x_ref, ew1_ref, eb1_ref, ew2_ref, eb2_ref, cb_ref,
          dw1_ref, db1_ref, dw2_ref, db2_ref,
          recon_ref, idx_ref, rl_ref, cl_ref, csq_ref,
          cbh_ref, cbm_ref, cbl_ref):
    i = pl.program_id(0)
    nsteps = pl.num_programs(0)

    @pl.when(i == 0)
    def _precompute():
        ones = jnp.ones((1, _HIDDEN), jnp.float32)
        for q in range(_Q):
            cb = cb_ref[q]
            csq_ref[q, 0:1, :] = jax.lax.dot_general(
                ones, cb * cb, (((1,), (1,)), ((), ())), precision=_HI)
            hi = cb.astype(jnp.bfloat16)
            r1 = cb - hi.astype(jnp.float32)
            mid = r1.astype(jnp.bfloat16)
            lo = (r1 - mid.astype(jnp.float32)).astype(jnp.bfloat16)
            cbh_ref[q] = hi
            cbm_ref[q] = mid
            cbl_ref[q] = lo
        rl_ref[...] = jnp.zeros((1, 1), jnp.float32)
        cl_ref[...] = jnp.zeros((1, 1), jnp.float32)

    x = x_ref[...]
    h = jnp.maximum(
        jax.lax.dot(x, ew1_ref[...], precision=_LO) + eb1_ref[...], 0.0)
    z = jax.lax.dot(h, ew2_ref[...], precision=_LO) + eb2_ref[...]

    _HB = _BLK // 2
    lane_q = jax.lax.broadcasted_iota(jnp.int32, (_HB, _Q), 1)

    def _half(q, r, qsum, commit, idx_acc):
        cb = cb_ref[q]
        rc = jax.lax.dot_general(
            r, cb, (((1,), (1,)), ((), ())), precision=_LO)
        rsq = jnp.sum(r * r, axis=1, keepdims=True)
        d = rsq - 2.0 * rc + csq_ref[q, 0:1, :]
        idx = jnp.argmin(d, axis=1)
        onehot = (jax.lax.broadcasted_iota(jnp.int32, (_HB, _K), 1)
                  == idx[:, None]).astype(jnp.bfloat16)
        quant = (jax.lax.dot(onehot, cbh_ref[q],
                             preferred_element_type=jnp.float32)
                 + jax.lax.dot(onehot, cbm_ref[q],
                               preferred_element_type=jnp.float32))
        diff = quant - r
        commit = commit + jnp.sum(diff * diff, keepdims=True)
        idx_acc = jnp.where(lane_q == q, idx[:, None], idx_acc)
        return r - quant, qsum + quant, commit, idx_acc

    def _level(q, carry):
        ra, rb, qsa, qsb, commit, ia, ib = carry
        ra, qsa, commit, ia = _half(q, ra, qsa, commit, ia)
        rb, qsb, commit, ib = _half(q, rb, qsb, commit, ib)
        return ra, rb, qsa, qsb, commit, ia, ib

    za, zb = z[:_HB, :], z[_HB:, :]
    zero_h = jnp.zeros((_HB, _HIDDEN), jnp.float32)
    zero_i = jnp.zeros((_HB, _Q), jnp.int32)
    ra, rb, qsa, qsb, commit, ia, ib = jax.lax.fori_loop(
        0, _Q,
        _level,
        (za, zb, zero_h, zero_h, jnp.zeros((1, 1), jnp.float32),
         zero_i, zero_i))
    qsum = jnp.concatenate([qsa, qsb], axis=0)

    idx_ref[...] = jnp.concatenate([ia, ib], axis=0)

    a = jnp.maximum(
        jax.lax.dot(qsum, dw1_ref[...], precision=_LO) + db1_ref[...], 0.0)
    recon = jax.lax.dot(a, dw2_ref[...], precision=_LO) + db2_ref[...]
    recon_ref[...] = recon

    rdiff = recon - x
    rsum = jnp.sum(rdiff * rdiff, keepdims=True)

    rl_ref[...] += rsum
    cl_ref[...] += commit

    @pl.when(i == nsteps - 1)
    def _finalize():
        rl_ref[...] = rl_ref[...] * (1.0 / (_B * _EMBED))
        cl_ref[...] = cl_ref[...] * (_CW / (_B * _HIDDEN))


@functools.partial(jax.jit, static_argnames=())
def kernel(x, enc_w1, enc_b1, enc_w2, enc_b2, codebooks,
           dec_w1, dec_b1, dec_w2, dec_b2):
    grid = (_B // _BLK,)
    full = lambda shape: pl.BlockSpec(shape, lambda i: (0,) * len(shape))
    recon, idxs, rl, cl = pl.pallas_call(
        _body,
        grid=grid,
        in_specs=[
            pl.BlockSpec((_BLK, _EMBED), lambda i: (i, 0)),
            full((_EMBED, _HIDDEN)),
            full((1, _HIDDEN)),
            full((_HIDDEN, _HIDDEN)),
            full((1, _HIDDEN)),
            full((_Q, _K, _HIDDEN)),
            full((_HIDDEN, _HIDDEN)),
            full((1, _HIDDEN)),
            full((_HIDDEN, _EMBED)),
            full((1, _EMBED)),
        ],
        out_specs=[
            pl.BlockSpec((_BLK, _EMBED), lambda i: (i, 0)),
            pl.BlockSpec((_BLK, _Q), lambda i: (i, 0)),
            pl.BlockSpec((1, 1), lambda i: (0, 0)),
            pl.BlockSpec((1, 1), lambda i: (0, 0)),
        ],
        out_shape=[
            jax.ShapeDtypeStruct((_B, _EMBED), jnp.float32),
            jax.ShapeDtypeStruct((_B, _Q), jnp.int32),
            jax.ShapeDtypeStruct((1, 1), jnp.float32),
            jax.ShapeDtypeStruct((1, 1), jnp.float32),
        ],
        scratch_shapes=[pltpu.VMEM((_Q, 8, _K), jnp.float32),
                        pltpu.VMEM((_Q, _K, _HIDDEN), jnp.bfloat16),
                        pltpu.VMEM((_Q, _K, _HIDDEN), jnp.bfloat16),
                        pltpu.VMEM((_Q, _K, _HIDDEN), jnp.bfloat16)],
    )(x, enc_w1, enc_b1[None, :], enc_w2, enc_b2[None, :], codebooks,
      dec_w1, dec_b1[None, :], dec_w2, dec_b2[None, :])
    return recon, idxs, rl[0, 0], cl[0, 0]
